# Initial kernel scaffold; baseline (speedup 1.0000x reference)
import jax
import jax.numpy as jnp
from jax.experimental import pallas as pl


def kernel(x, edge_index, edge_attr, W_lin, b_lin, W_edge, b_edge):
    raise NotImplementedError("write your pallas kernel here")


# speedup vs baseline: 4.4686x; 4.4686x over previous
import functools

import jax
import jax.numpy as jnp
from jax import lax
from jax.experimental import pallas as pl
from jax.experimental.pallas import tpu as pltpu
from jax.experimental.pallas import tpu_sc as plsc

N = 10000
D = 128
DE = 16
NC, NS = 2, 16
NW = NC * NS
CH = 128
RPW = 640
R = NS * RPW
LANES = 16


def _lin_body(x_ref, w_ref, b_ref, o_ref):
    o_ref[...] = (
        jnp.dot(x_ref[...], w_ref[...], preferred_element_type=jnp.float32)
        + b_ref[...]
    )


def _node_linear(x, w, b):
    blk = 2000
    return pl.pallas_call(
        _lin_body,
        grid=(N // blk,),
        in_specs=[
            pl.BlockSpec((blk, D), lambda i: (i, 0)),
            pl.BlockSpec((D, D), lambda i: (0, 0)),
            pl.BlockSpec((D,), lambda i: (0,)),
        ],
        out_specs=pl.BlockSpec((blk, D), lambda i: (i, 0)),
        out_shape=jax.ShapeDtypeStruct((N, D), jnp.float32),
    )(x, w, b)


def _sc_rows_body(xl_hbm, src_hbm, dst_hbm, z_big, agg_hbm,
                  src_v, dst_v, rows_v, sem, acc, n_chunks):
    c = lax.axis_index("c")
    s = lax.axis_index("s")
    wid = s * NC + c
    row0 = s * RPW
    n_sub = RPW // CH

    pltpu.sync_copy(z_big, rows_v)
    for r in range(n_sub):
        pltpu.sync_copy(rows_v, acc.at[pl.ds(row0 + r * CH, CH)])
    plsc.subcore_barrier()

    base = wid * (n_chunks * CH)

    def chunk(j, carry):
        off = base + j * CH
        pltpu.sync_copy(src_hbm.at[pl.ds(off, CH)], src_v)
        pltpu.sync_copy(dst_hbm.at[pl.ds(off, CH)], dst_v)
        pltpu.async_copy(xl_hbm.at[src_v], rows_v, sem).wait()
        pltpu.sync_copy(rows_v, acc.at[dst_v], add=True)
        return carry

    lax.fori_loop(0, n_chunks, chunk, 0)
    plsc.subcore_barrier()

    out0 = c * R + row0
    for r in range(n_sub):
        pltpu.sync_copy(acc.at[pl.ds(row0 + r * CH, CH)], rows_v)
        pltpu.sync_copy(rows_v, agg_hbm.at[pl.ds(out0 + r * CH, CH)])


def _sc_rows(xl, src_p, dst_p, n_chunks):
    mesh = plsc.VectorSubcoreMesh(core_axis_name="c", subcore_axis_name="s")
    kfn = functools.partial(
        pl.kernel,
        out_type=jax.ShapeDtypeStruct((NC * R, D), jnp.float32),
        mesh=mesh,
        scratch_types=[
            pltpu.VMEM((CH,), jnp.int32),
            pltpu.VMEM((CH,), jnp.int32),
            pltpu.VMEM((CH, D), jnp.float32),
            pltpu.SemaphoreType.DMA,
            pltpu.VMEM_SHARED((R, D), jnp.float32),
        ],
    )(functools.partial(_sc_rows_body, n_chunks=n_chunks))
    return kfn(xl, src_p, dst_p, jnp.zeros((CH, D), jnp.float32))


CH2 = 2048


def _sc_ea_body(ea_hbm, dst_hbm, sat_hbm, dst_v, val_v, acc2, n_chunks2):
    c = lax.axis_index("c")
    s = lax.axis_index("s")
    half = n_chunks2 * CH2

    def zero(i, carry):
        acc2[pl.ds(i * LANES, LANES)] = jnp.zeros((LANES,), jnp.float32)
        return carry

    lax.fori_loop(0, R // LANES, zero, 0)

    def chunk(j, carry):
        off = c * half + j * CH2
        pltpu.sync_copy(dst_hbm.at[pl.ds(off, CH2)], dst_v)
        pltpu.sync_copy(ea_hbm.at[s, pl.ds(off, CH2)], val_v)
        for i in range(CH2 // LANES):
            idx = dst_v[pl.ds(i * LANES, LANES)]
            val = val_v[pl.ds(i * LANES, LANES)]
            plsc.addupdate_scatter(acc2, [idx], val)
        return carry

    lax.fori_loop(0, n_chunks2, chunk, 0)
    pltpu.sync_copy(acc2, sat_hbm.at[pl.ds((c * DE + s) * R, R)])


def _sc_ea(ea_t, dst_p, n_chunks2):
    mesh = plsc.VectorSubcoreMesh(core_axis_name="c", subcore_axis_name="s")
    kfn = functools.partial(
        pl.kernel,
        out_type=jax.ShapeDtypeStruct((NC * DE * R,), jnp.float32),
        mesh=mesh,
        compiler_params=pltpu.CompilerParams(needs_layout_passes=False),
        scratch_types=[
            pltpu.VMEM((CH2,), jnp.int32),
            pltpu.VMEM((CH2,), jnp.float32),
            pltpu.VMEM((R,), jnp.float32),
        ],
    )(functools.partial(_sc_ea_body, n_chunks2=n_chunks2))
    return kfn(ea_t, dst_p)


def _comb_body(p_ref, t_ref, xl_ref, w_ref, o_ref):
    svec = t_ref[0] + t_ref[1]
    o_ref[...] = (
        p_ref[0]
        + p_ref[1]
        + xl_ref[...]
        + jnp.dot(svec, w_ref[...], preferred_element_type=jnp.float32)
        + jnp.sum(w_ref[...], axis=0)[None, :]
    )


def _combine(agg, sat, xl, w_ext):
    blk = 2000
    return pl.pallas_call(
        _comb_body,
        grid=(N // blk,),
        in_specs=[
            pl.BlockSpec((NC, blk, D), lambda i: (0, i, 0)),
            pl.BlockSpec((NC, blk, DE), lambda i: (0, i, 0)),
            pl.BlockSpec((blk, D), lambda i: (i, 0)),
            pl.BlockSpec((DE, D), lambda i: (0, 0)),
        ],
        out_specs=pl.BlockSpec((blk, D), lambda i: (i, 0)),
        out_shape=jax.ShapeDtypeStruct((N, D), jnp.float32),
    )(agg, sat, xl, w_ext)


def kernel(x, edge_index, edge_attr, W_lin, b_lin, W_edge, b_edge):
    E = edge_index.shape[1]
    n_chunks = -(-E // (NW * CH))
    EP = NW * CH * n_chunks
    src = edge_index[0].astype(jnp.int32)
    dst = edge_index[1].astype(jnp.int32)
    pad = EP - E
    src_p = jnp.concatenate([src, jnp.zeros((pad,), jnp.int32)])
    dst_p = jnp.concatenate([dst, jnp.full((pad,), N, jnp.int32)])
    ea_ext = jnp.concatenate(
        [edge_attr,
         jnp.ones((E, 1), jnp.float32),
         jnp.zeros((E, DE - edge_attr.shape[1] - 1), jnp.float32)], axis=1)
    ea_t = jnp.concatenate(
        [ea_ext, jnp.zeros((pad, DE), jnp.float32)], axis=0).T
    w_ext = jnp.concatenate(
        [W_edge, b_edge[None, :],
         jnp.zeros((DE - W_edge.shape[0] - 1, D), jnp.float32)], axis=0)

    xl = _node_linear(x, W_lin, b_lin)
    agg = _sc_rows(xl, src_p, dst_p, n_chunks)
    n_chunks2 = EP // (NC * CH2)
    sat = _sc_ea(ea_t, dst_p, n_chunks2)
    agg = agg.reshape(NC, R, D)[:, :N, :]
    sat = sat.reshape(NC, DE, R)[:, :, :N].transpose(0, 2, 1)
    return _combine(agg, sat, xl, w_ext)
